# Initial kernel scaffold; baseline (speedup 1.0000x reference)
import jax
import jax.numpy as jnp
from jax.experimental import pallas as pl


def kernel(xyzt, canon_table, xyt_table, yzt_table, xzt_table):
    raise NotImplementedError("write your pallas kernel here")


# speedup vs baseline: 7.5765x; 7.5765x over previous
import functools

import numpy as np
import jax
import jax.numpy as jnp
from jax import lax
from jax.experimental import pallas as pl
from jax.experimental.pallas import tpu as pltpu
from jax.experimental.pallas import tpu_sc as plsc

_BOUND = 1.6
_LD = 2
_PRIME1 = np.int32(2654435761 - (1 << 32))
_PRIME2 = np.int32(805459861)
_HMASK = np.int32((1 << 19) - 1)
_NW = 32


def _mk_levels(num_levels, base_resolution, desired_resolution, log2_hashmap_size, input_dim=3):
    base = np.broadcast_to(np.asarray(base_resolution, dtype=np.float64), (input_dim,)).astype(np.float64)
    desired = np.broadcast_to(np.asarray(desired_resolution, dtype=np.float64), (input_dim,)).astype(np.float64)
    if num_levels > 1:
        scale = np.exp((np.log(desired) - np.log(base)) / (num_levels - 1))
    else:
        scale = np.ones(input_dim)
    max_params = 2 ** log2_hashmap_size
    levels = []
    offset = 0
    for l in range(num_levels):
        res = np.maximum(np.ceil(base * (scale ** l) - 1e-6).astype(np.int64), 1)
        n_grid = int(np.prod(res + 1))
        size = int(min(max_params, n_grid))
        levels.append((tuple(int(r) for r in res), size, offset, n_grid > max_params))
        offset += size
    return levels, offset


_CANON, _CANON_TOTAL = _mk_levels(16, 16, 2048, 19)
_DEFORM, _DEFORM_TOTAL = _mk_levels(32, [8, 8, 8], [32, 32, 16], 19)

_BUCKETS = [(0, 16), (16, 24), (24, 30), (30, 32)]
_ENC_DIMS = ((0, 1, 3), (1, 2, 3), (0, 2, 3))


def _bucket_stage_rows(lo, hi):
    mx = max(_DEFORM[l][1] for l in range(lo, hi))
    return ((mx + 15) // 8) * 8


_STAGE_ROWS = [_bucket_stage_rows(lo, hi) for lo, hi in _BUCKETS]
_TBL_ROWS = max(_STAGE_ROWS)
_PAD_ROWS = _TBL_ROWS + 16


def _deform_consts():
    ints, flts, ranges = [], [], []
    p = 0
    for (lo, hi) in _BUCKETS:
        start = p
        for e in range(3):
            for l in range(lo, hi):
                res, size, off, _ = _DEFORM[l]
                row_off = e * _DEFORM_TOTAL + off
                aoff = row_off & ~7
                delta = row_off - aoff
                s1 = res[0] + 1
                s2 = (res[0] + 1) * (res[1] + 1)
                d0, d1, d2 = _ENC_DIMS[e]
                orow = e * 64 + 2 * l
                ints.append([aoff, delta, s1, s2, res[0] - 1, res[1] - 1, res[2] - 1,
                             d0, d1, d2, orow] + [0] * 5)
                flts.append([float(res[0]), float(res[1]), float(res[2])] + [0.0] * 13)
                p += 1
        ranges.append((start, p))
    return (np.asarray(ints, np.int32), np.asarray(flts, np.float32), ranges)


_DCI, _DCF, _DRANGES = _deform_consts()

_CCI = np.asarray(
    [[lv[2], lv[0][0] + 1, (lv[0][0] + 1) ** 2, lv[0][0] - 1] + [0] * 12
     for lv in _CANON], np.int32)
_CCF = np.asarray([[float(lv[0][0])] + [0.0] * 15 for lv in _CANON], np.float32)
_N_DENSE_CANON = sum(1 for lv in _CANON if not lv[3])


def _lerp(a, b, t):
    return a + (b - a) * t


@functools.lru_cache(maxsize=4)
def _build(n_points):
    assert n_points % _NW == 0
    chunk = n_points // _NW
    q = min(512, chunk)
    assert q % 128 == 0 and chunk % q == 0
    nq = chunk // q
    nvq = q // 16
    nch = (q * 8) // 128
    rpc = q // 128
    nv = chunk // 16

    f32 = jnp.float32
    i32 = jnp.int32

    mesh = plsc.VectorSubcoreMesh(
        core_axis_name="c", subcore_axis_name="s", num_cores=2, num_subcores=16)

    def body(xyzt_h, canon_h, big_h, dci_h, dcf_h, cci_h, ccf_h,
             sp_h, tp_h,
             xyzt_v, u4, tbl, outb, idxb, lowb, rowsb, fracb,
             dci_v, dcf_v, cci_v, ccf_v, dsem):
        wid = lax.axis_index("s") * 2 + lax.axis_index("c")
        base = wid * chunk
        iota = lax.iota(i32, 16)
        z16 = iota * 0
        o16 = z16 + 1

        pltpu.sync_copy(dci_h, dci_v)
        pltpu.sync_copy(dcf_h, dcf_v)
        pltpu.sync_copy(cci_h, cci_v)
        pltpu.sync_copy(ccf_h, ccf_v)
        pltpu.sync_copy(xyzt_h.at[pl.ds(base * 4, chunk * 4)], xyzt_v)

        def u_body(v, _):
            p0 = v * 16
            w0 = (p0 + iota) * 4
            for d in range(4):
                x = plsc.load_gather(xyzt_v, [w0 + d])
                u = jnp.minimum(jnp.maximum((x + _BOUND) / (2.0 * _BOUND), 0.0), 1.0)
                u4[d, pl.ds(p0, 16)] = u
            return 0
        lax.fori_loop(0, nv, u_body, 0)

        def run_bucket(p_lo, p_hi, stage_rows):
            def pair_body(j, _):
                civ = dci_v[j, pl.ds(0, 16)]
                cfv = dcf_v[j, pl.ds(0, 16)]
                aoff = pl.multiple_of(civ[0], 8)
                delta = civ[1]
                s1 = civ[2]
                s2 = civ[3]
                i0m = civ[4]
                i1m = civ[5]
                i2m = civ[6]
                d0 = civ[7]
                d1 = civ[8]
                d2 = civ[9]
                orow = civ[10]
                r0 = cfv[0]
                r1 = cfv[1]
                r2 = cfv[2]
                pltpu.sync_copy(big_h.at[pl.ds(aoff * 2, stage_rows * 2)],
                                tbl.at[pl.ds(0, stage_rows * 2)])

                def vec(v, _):
                    p0 = v * 16
                    u0 = u4[d0, pl.ds(p0, 16)]
                    u1 = u4[d1, pl.ds(p0, 16)]
                    u2 = u4[d2, pl.ds(p0, 16)]
                    pos0 = u0 * r0
                    pos1 = u1 * r1
                    pos2 = u2 * r2
                    i0 = jnp.minimum(pos0.astype(i32), i0m)
                    i1 = jnp.minimum(pos1.astype(i32), i1m)
                    i2 = jnp.minimum(pos2.astype(i32), i2m)
                    f0 = pos0 - i0.astype(f32)
                    f1 = pos1 - i1.astype(f32)
                    f2 = pos2 - i2.astype(f32)
                    w000 = (i0 + i1 * s1 + i2 * s2 + delta) * 2
                    ga = []
                    gb = []
                    for bz in range(2):
                        for by in range(2):
                            for bx in range(2):
                                w = w000 + (bx + by * s1 + bz * s2) * 2
                                ga.append(plsc.load_gather(tbl, [w]))
                                gb.append(plsc.load_gather(tbl, [w + 1]))
                    acc0, acc1 = None, None
                    for g, out_i in ((ga, 0), (gb, 1)):
                        x00 = _lerp(g[0], g[1], f0)
                        x10 = _lerp(g[2], g[3], f0)
                        x01 = _lerp(g[4], g[5], f0)
                        x11 = _lerp(g[6], g[7], f0)
                        y0 = _lerp(x00, x10, f1)
                        y1 = _lerp(x01, x11, f1)
                        outb[out_i, pl.ds(p0, 16)] = _lerp(y0, y1, f2)
                    return 0
                lax.fori_loop(0, nv, vec, 0)
                pltpu.sync_copy(outb, tp_h.at[pl.ds(orow, 2), pl.ds(base, chunk)])
                return 0
            lax.fori_loop(p_lo, p_hi, pair_body, 0)

        for (p_lo, p_hi), srows in zip(_DRANGES, _STAGE_ROWS):
            run_bucket(p_lo, p_hi, srows)

        def canon_pass2(lvl, qi):
            def vec(v, _):
                p0 = qi * q + v * 16
                f0 = fracb[0, pl.ds(v * 16, 16)]
                f1 = fracb[1, pl.ds(v * 16, 16)]
                f2 = fracb[2, pl.ds(v * 16, 16)]
                jrow0 = v // 8
                col0 = (v % 8) * 16
                cols = col0 + iota
                ga = []
                gb = []
                for c in range(8):
                    j = c * rpc + jrow0
                    low = lowb[j, pl.ds(col0, 16)]
                    ga.append(plsc.load_gather(rowsb, [z16 + j, cols, low]))
                    gb.append(plsc.load_gather(rowsb, [z16 + j, cols, low + 1]))
                for g, out_i in ((ga, 0), (gb, 1)):
                    x00 = _lerp(g[0], g[1], f0)
                    x10 = _lerp(g[2], g[3], f0)
                    x01 = _lerp(g[4], g[5], f0)
                    x11 = _lerp(g[6], g[7], f0)
                    y0 = _lerp(x00, x10, f1)
                    y1 = _lerp(x01, x11, f1)
                    outb[out_i, pl.ds(p0, 16)] = _lerp(y0, y1, f2)
                return 0
            lax.fori_loop(0, nvq, vec, 0)

        def canon_level(lvl, hashed):
            civ = cci_v[lvl, pl.ds(0, 16)]
            off = civ[0]
            s1 = civ[1]
            s2 = civ[2]
            rm1 = civ[3]
            rf = ccf_v[lvl, pl.ds(0, 16)][0]

            def quarter(qi, _):
                def pass1(v, _):
                    p0 = qi * q + v * 16
                    jrow0 = v // 8
                    col0 = (v % 8) * 16
                    i_ = []
                    fr = []
                    for d in range(3):
                        u = u4[d, pl.ds(p0, 16)]
                        pos = u * rf
                        ii = jnp.minimum(pos.astype(i32), rm1)
                        i_.append(ii)
                        fr.append(pos - ii.astype(f32))
                    for d in range(3):
                        fracb[d, pl.ds(v * 16, 16)] = fr[d]
                    if hashed:
                        hx0 = i_[0]
                        hx1 = i_[0] + 1
                        hy0 = i_[1] * _PRIME1
                        hy1 = hy0 + _PRIME1
                        hz0 = i_[2] * _PRIME2
                        hz1 = hz0 + _PRIME2
                        exy = [hx0 ^ hy0, hx1 ^ hy0, hx0 ^ hy1, hx1 ^ hy1]
                        for c in range(8):
                            hz = hz1 if (c >> 2) & 1 else hz0
                            r = ((exy[c & 3] ^ hz) & _HMASK) + off
                            idxb[c * rpc + jrow0, pl.ds(col0, 16)] = r >> 2
                            lowb[c * rpc + jrow0, pl.ds(col0, 16)] = (r & 3) * 2
                    else:
                        b000 = i_[0] + i_[1] * s1 + i_[2] * s2 + off
                        for c in range(8):
                            r = b000 + (c & 1) + ((c >> 1) & 1) * s1 + ((c >> 2) & 1) * s2
                            idxb[c * rpc + jrow0, pl.ds(col0, 16)] = r >> 2
                            lowb[c * rpc + jrow0, pl.ds(col0, 16)] = (r & 3) * 2
                    return 0
                lax.fori_loop(0, nvq, pass1, 0)

                def fire(j, _):
                    pltpu.async_copy(canon_h.at[idxb.at[j]], rowsb.at[j], dsem)
                    return 0
                lax.fori_loop(0, nch, fire, 0)

                def drain(j, _):
                    pltpu.make_async_copy(canon_h.at[idxb.at[j]], rowsb.at[j], dsem).wait()
                    return 0
                lax.fori_loop(0, nch, drain, 0)

                canon_pass2(lvl, qi)
                return 0
            lax.fori_loop(0, nq, quarter, 0)
            pltpu.sync_copy(outb, sp_h.at[pl.ds(2 * lvl, 2), pl.ds(base, chunk)])

        def canon_dense(lvl, _):
            canon_level(lvl, False)
            return 0
        lax.fori_loop(0, _N_DENSE_CANON, canon_dense, 0)

        def canon_hash(lvl, _):
            canon_level(lvl, True)
            return 0
        lax.fori_loop(_N_DENSE_CANON, 16, canon_hash, 0)

    grid_kernel = pl.kernel(
        body,
        out_type=(
            jax.ShapeDtypeStruct((32, n_points), jnp.float32),
            jax.ShapeDtypeStruct((192, n_points), jnp.float32),
        ),
        mesh=mesh,
        scratch_types=[
            pltpu.VMEM((chunk * 4,), f32),
            pltpu.VMEM((4, chunk), f32),
            pltpu.VMEM((_TBL_ROWS * 2,), f32),
            pltpu.VMEM((2, chunk), f32),
            pltpu.VMEM((nch, 128), i32),
            pltpu.VMEM((nch, 128), i32),
            pltpu.VMEM((nch, 128, 8), f32),
            pltpu.VMEM((3, q), f32),
            pltpu.VMEM((96, 16), i32),
            pltpu.VMEM((96, 16), f32),
            pltpu.VMEM((16, 16), i32),
            pltpu.VMEM((16, 16), f32),
            pltpu.SemaphoreType.DMA,
        ],
        compiler_params=pltpu.CompilerParams(
            use_tc_tiling_on_sc=False, needs_layout_passes=False),
    )
    return grid_kernel


_CANON_VIEW_ROWS = (_CANON_TOTAL * _LD + 6 + 7) // 8


def kernel(xyzt, canon_table, xyt_table, yzt_table, xzt_table):
    n = xyzt.shape[0]
    big = jnp.concatenate(
        [xyt_table, yzt_table, xzt_table,
         jnp.zeros((_PAD_ROWS, _LD), jnp.float32)], axis=0).reshape(-1)
    cflat = canon_table.reshape(-1)
    canon8 = jnp.concatenate(
        [cflat, jnp.zeros((_CANON_VIEW_ROWS * 8 - cflat.shape[0],), jnp.float32)]
    ).reshape(_CANON_VIEW_ROWS, 8)
    sp_t, tp_t = _build(n)(
        xyzt.reshape(-1), canon8, big,
        jnp.asarray(_DCI), jnp.asarray(_DCF),
        jnp.asarray(_CCI), jnp.asarray(_CCF))
    return sp_t.T, tp_t.T
